# Initial kernel scaffold; baseline (speedup 1.0000x reference)
import jax
import jax.numpy as jnp
from jax.experimental import pallas as pl


def kernel(queries, keys, k):
    raise NotImplementedError("write your pallas kernel here")


# speedup vs baseline: 4.5108x; 4.5108x over previous
import functools

import jax
import jax.numpy as jnp
from jax.experimental import pallas as pl
from jax.experimental.pallas import tpu as pltpu

_B = 16384
_L = 128


def _p1_body(nkeys, q_ref, k_ref, s_ref, m_ref):
    i = pl.program_id(0)
    s = jax.lax.dot_general(q_ref[...], k_ref[...], (((1,), (1,)), ((), ())),
                            preferred_element_type=jnp.float32)
    col = i * _B + jax.lax.broadcasted_iota(jnp.int32, s.shape, 1)
    s = jnp.where(col < nkeys, s, -jnp.inf)
    s_ref[...] = s
    m_ref[...] = jnp.max(s.reshape(s.shape[0], _B // _L, _L), axis=2)


def _phase1(queries, keys):
    q, d = queries.shape
    n = keys.shape[0]
    nb = pl.cdiv(n, _B)
    npad = nb * _B
    s, m = pl.pallas_call(
        functools.partial(_p1_body, n),
        grid=(nb,),
        in_specs=[
            pl.BlockSpec((q, d), lambda i: (0, 0)),
            pl.BlockSpec((_B, d), lambda i: (i, 0)),
        ],
        out_specs=[
            pl.BlockSpec((q, _B), lambda i: (0, i)),
            pl.BlockSpec((q, _B // _L), lambda i: (0, i)),
        ],
        out_shape=[
            jax.ShapeDtypeStruct((q, npad), jnp.float32),
            jax.ShapeDtypeStruct((q, npad // _L), jnp.float32),
        ],
    )(queries, keys)
    return s, m


def kernel(queries, keys, k):
    q = queries.shape[0]
    kk = q
    scores, mx = _phase1(queries, keys)
    g = mx.shape[1]
    table = scores.reshape(q * g, _L)

    _, top_groups = jax.lax.top_k(mx, kk)
    top_groups = jnp.sort(top_groups, axis=1)
    rowids = jnp.arange(q)[:, None] * g + top_groups
    cand = table[rowids]
    cand_idx = top_groups[:, :, None] * _L + jnp.arange(_L)[None, None, :]
    cand = cand.reshape(q, kk * _L)
    cand_idx = cand_idx.reshape(q, kk * _L)
    top_vals, pos = jax.lax.top_k(cand, kk)
    top_idx = jnp.take_along_axis(cand_idx, pos, axis=1) + (k - kk)
    neighbours = jnp.take(keys, top_idx, axis=0)
    return top_vals, top_idx, neighbours


# speedup vs baseline: 5.7504x; 1.2748x over previous
import functools

import jax
import jax.numpy as jnp
from jax import lax
from jax.experimental import pallas as pl
from jax.experimental.pallas import tpu as pltpu
from jax.experimental.pallas import tpu_sc as plsc

_B = 16384
_L = 128
_BIG = float(1 << 30)
_NEG = float("-inf")


def _i32(x):
    return lax.convert_element_type(x, jnp.int32)


def _p1_body(nkeys, q_ref, k_ref, s_ref, m_ref):
    i = pl.program_id(0)
    s = lax.dot_general(q_ref[...], k_ref[...], (((1,), (1,)), ((), ())),
                        preferred_element_type=jnp.float32)
    col = i * _B + lax.broadcasted_iota(jnp.int32, s.shape, 1)
    s = jnp.where(col < nkeys, s, _NEG)
    s_ref[...] = s
    m_ref[...] = jnp.max(s.reshape(s.shape[0], _B // _L, _L), axis=2)


def _phase1(queries, keys):
    q, d = queries.shape
    n = keys.shape[0]
    nb = pl.cdiv(n, _B)
    npad = nb * _B
    return pl.pallas_call(
        functools.partial(_p1_body, n),
        grid=(nb,),
        in_specs=[
            pl.BlockSpec((q, d), lambda i: (0, 0)),
            pl.BlockSpec((_B, d), lambda i: (i, 0)),
        ],
        out_specs=[
            pl.BlockSpec((q, _B), lambda i: (0, i)),
            pl.BlockSpec((q, _B // _L), lambda i: (0, i)),
        ],
        out_shape=[
            jax.ShapeDtypeStruct((q, npad), jnp.float32),
            jax.ShapeDtypeStruct((q, npad // _L), jnp.float32),
        ],
    )(queries, keys)


def _iota16():
    return lax.broadcasted_iota(jnp.int32, (16,), 0)


def _vset(ref, pos, val):
    base = (pos // 16) * 16
    v = ref[pl.ds(base, 16)]
    ref[pl.ds(base, 16)] = jnp.where(_iota16() == pos % 16, val, v)


def _sc_phase2(mx, table, keys, shift, kk):
    qn, g = mx.shape
    nkr = keys.shape[0]
    d = keys.shape[1]
    rows = g // 16
    rpad = ((rows + 15) // 16) * 16
    nsup = rpad // 16
    spad = ((nsup + 15) // 16) * 16
    ncand = kk * _L
    cchunks = ncand // 16
    csup_n = cchunks // 16
    mesh = plsc.VectorSubcoreMesh(core_axis_name="c", subcore_axis_name="s")

    @functools.partial(
        pl.kernel,
        out_type=[
            jax.ShapeDtypeStruct((qn, kk), jnp.float32),
            jax.ShapeDtypeStruct((qn, kk), jnp.int32),
            jax.ShapeDtypeStruct((qn, kk, d), jnp.float32),
        ],
        mesh=mesh,
        compiler_params=pltpu.CompilerParams(needs_layout_passes=False),
        scratch_types=[
            pltpu.VMEM((g,), jnp.float32),
            pltpu.VMEM((rpad,), jnp.float32),
            pltpu.VMEM((spad,), jnp.float32),
            pltpu.VMEM((kk,), jnp.float32),
            pltpu.VMEM((kk,), jnp.int32),
            pltpu.VMEM((kk, _L), jnp.float32),
            pltpu.VMEM((cchunks,), jnp.float32),
            pltpu.VMEM((16,), jnp.float32),
            pltpu.VMEM((kk,), jnp.float32),
            pltpu.VMEM((kk,), jnp.int32),
            pltpu.VMEM((kk, d), jnp.float32),
            pltpu.VMEM((16,), jnp.float32),
            pltpu.SMEM((kk,), jnp.int32),
            pltpu.SemaphoreType.DMA,
        ],
    )
    def body(mx_hbm, table_hbm, keys_hbm, shift_hbm, tv_hbm, ti_hbm, nb_hbm,
             mx_v, rowmax_v, sup_v, gsel_v, grow_v, cand_v, crow_v, csup_v,
             vals_v, idx_v, nbr_v, shift_v, gs_s, sem):
        wid = lax.axis_index("s") * 2 + lax.axis_index("c")
        pltpu.sync_copy(mx_hbm.at[wid], mx_v)
        pltpu.sync_copy(shift_hbm, shift_v)
        shift = _i32(jnp.max(shift_v[...]))
        iota = _iota16()
        fiota = lax.convert_element_type(iota, jnp.float32)

        for vb in range(rpad // 16):
            rowmax_v[pl.ds(vb * 16, 16)] = jnp.full((16,), _NEG, jnp.float32)

        def rm_body(vb, _):
            vec = jnp.full((16,), _NEG, jnp.float32)
            for l in range(16):
                m = jnp.max(mx_v[pl.ds(vb * 256 + l * 16, 16)])
                vec = jnp.where(iota == l, m, vec)
            rowmax_v[pl.ds(vb * 16, 16)] = vec
            return 0

        lax.fori_loop(0, rows // 16, rm_body, 0)
        if rows % 16:
            vb0 = rows // 16
            vec = jnp.full((16,), _NEG, jnp.float32)
            for l in range(rows % 16):
                m = jnp.max(mx_v[pl.ds(vb0 * 256 + l * 16, 16)])
                vec = jnp.where(iota == l, m, vec)
            rowmax_v[pl.ds(vb0 * 16, 16)] = vec

        for vb in range(spad // 16):
            sup_v[pl.ds(vb * 16, 16)] = jnp.full((16,), _NEG, jnp.float32)

        def sup_body(vb, _):
            _vset(sup_v, vb, jnp.max(rowmax_v[pl.ds(vb * 16, 16)]))
            return 0

        lax.fori_loop(0, nsup, sup_body, 0)

        def sel_body(j, _):
            s0 = sup_v[pl.ds(0, 16)]
            s1 = sup_v[pl.ds(16, 16)]
            m0 = jnp.max(jnp.maximum(s0, s1))
            vb = _i32(jnp.minimum(
                jnp.min(jnp.where(s0 == m0, fiota, _BIG)),
                jnp.min(jnp.where(s1 == m0, fiota + 16.0, _BIG))))
            rv = rowmax_v[pl.ds(vb * 16, 16)]
            r = vb * 16 + _i32(jnp.min(jnp.where(rv == m0, fiota, _BIG)))
            w = mx_v[pl.ds(r * 16, 16)]
            lane = _i32(jnp.min(jnp.where(w == m0, fiota, _BIG)))
            gid = r * 16 + lane
            w2 = jnp.where(iota == lane, _NEG, w)
            mx_v[pl.ds(r * 16, 16)] = w2
            rv2 = jnp.where(iota == r % 16, jnp.max(w2), rv)
            rowmax_v[pl.ds(vb * 16, 16)] = rv2
            _vset(sup_v, vb, jnp.max(rv2))
            _vset(gsel_v, j, lax.convert_element_type(gid, jnp.float32))
            return 0

        lax.fori_loop(0, kk, sel_body, 0)

        def sort_body(a, _):
            g0 = gsel_v[pl.ds(0, 16)]
            g1 = gsel_v[pl.ds(16, 16)]
            m = jnp.minimum(jnp.min(g0), jnp.min(g1))
            p = _i32(jnp.minimum(
                jnp.min(jnp.where(g0 == m, fiota, _BIG)),
                jnp.min(jnp.where(g1 == m, fiota + 16.0, _BIG))))
            mi = _i32(m)
            _vset(gsel_v, p, _BIG)
            _vset(grow_v, a, wid * g + mi)
            gs_s[a] = mi
            return 0

        lax.fori_loop(0, kk, sort_body, 0)

        pltpu.async_copy(table_hbm.at[grow_v], cand_v, sem).wait()

        def cr_body(vb, _):
            vec = jnp.full((16,), _NEG, jnp.float32)
            for l in range(16):
                m = jnp.max(cand_v[vb * 2 + (l // 8), pl.ds((l % 8) * 16, 16)])
                vec = jnp.where(iota == l, m, vec)
            crow_v[pl.ds(vb * 16, 16)] = vec
            _vset(csup_v, vb, jnp.max(vec))
            return 0

        lax.fori_loop(0, csup_n, cr_body, 0)

        def fin_body(j, _):
            c0 = csup_v[...]
            m0 = jnp.max(c0)
            vb = _i32(jnp.min(jnp.where(c0 == m0, fiota, _BIG)))
            rv = crow_v[pl.ds(vb * 16, 16)]
            t = vb * 16 + _i32(jnp.min(jnp.where(rv == m0, fiota, _BIG)))
            a = t // 8
            sub = (t % 8) * 16
            w = cand_v[a, pl.ds(sub, 16)]
            lane = _i32(jnp.min(jnp.where(w == m0, fiota, _BIG)))
            p = t * 16 + lane
            glob = gs_s[p // _L] * _L + p % _L + shift
            w2 = jnp.where(iota == lane, _NEG, w)
            cand_v[a, pl.ds(sub, 16)] = w2
            rv2 = jnp.where(iota == t % 16, jnp.max(w2), rv)
            crow_v[pl.ds(vb * 16, 16)] = rv2
            _vset(csup_v, vb, jnp.max(rv2))
            _vset(vals_v, j, m0)
            _vset(idx_v, j, glob)
            return 0

        lax.fori_loop(0, kk, fin_body, 0)

        pltpu.async_copy(keys_hbm.at[idx_v], nbr_v, sem).wait()
        pltpu.sync_copy(vals_v, tv_hbm.at[wid])
        pltpu.sync_copy(idx_v, ti_hbm.at[wid])
        pltpu.sync_copy(nbr_v, nb_hbm.at[wid])

    return body(mx, table, keys, shift)


def kernel(queries, keys, k):
    q = queries.shape[0]
    kk = q
    scores, mx = _phase1(queries, keys)
    table = scores.reshape(-1, _L)
    shift = jnp.full((16,), k - kk, jnp.float32)
    top_vals, top_idx, neighbours = _sc_phase2(mx, table, keys, shift, kk)
    return top_vals, top_idx, neighbours


# speedup vs baseline: 10.4689x; 1.8206x over previous
import functools

import jax
import jax.numpy as jnp
from jax import lax
from jax.experimental import pallas as pl
from jax.experimental.pallas import tpu as pltpu
from jax.experimental.pallas import tpu_sc as plsc

_B = 16384
_L = 128
_BIG = float(1 << 30)
_NEG = float("-inf")


def _i32(x):
    return lax.convert_element_type(x, jnp.int32)


def _p1_body(nkeys, q_ref, k_ref, s_ref, m_ref):
    i = pl.program_id(0)
    s = lax.dot_general(q_ref[...], k_ref[...], (((1,), (1,)), ((), ())),
                        preferred_element_type=jnp.float32)
    col = i * _B + lax.broadcasted_iota(jnp.int32, s.shape, 1)
    s = jnp.where(col < nkeys, s, _NEG)
    s_ref[...] = s
    m_ref[...] = jnp.max(s.reshape(s.shape[0], _B // _L, _L), axis=2)


def _phase1(queries, keys):
    q, d = queries.shape
    n = keys.shape[0]
    nb = pl.cdiv(n, _B)
    npad = nb * _B
    return pl.pallas_call(
        functools.partial(_p1_body, n),
        grid=(nb,),
        in_specs=[
            pl.BlockSpec((q, d), lambda i: (0, 0)),
            pl.BlockSpec((_B, d), lambda i: (i, 0)),
        ],
        out_specs=[
            pl.BlockSpec((q, _B), lambda i: (0, i)),
            pl.BlockSpec((q, _B // _L), lambda i: (0, i)),
        ],
        out_shape=[
            jax.ShapeDtypeStruct((q, npad), jnp.float32),
            jax.ShapeDtypeStruct((q, npad // _L), jnp.float32),
        ],
    )(queries, keys)


def _iota16():
    return lax.broadcasted_iota(jnp.int32, (16,), 0)


def _vset(ref, pos, val):
    base = (pos // 16) * 16
    v = ref[pl.ds(base, 16)]
    ref[pl.ds(base, 16)] = jnp.where(_iota16() == pos % 16, val, v)


def _sc_phase2(mx, scores, keys, shift, kk):
    qn, g = mx.shape
    nkr = keys.shape[0]
    d = keys.shape[1]
    rows = g // 16
    rpad = ((rows + 15) // 16) * 16
    nsup = rpad // 16
    spad = ((nsup + 15) // 16) * 16
    ncand = kk * _L
    cchunks = ncand // 16
    csup_n = cchunks // 16
    mesh = plsc.VectorSubcoreMesh(core_axis_name="c", subcore_axis_name="s")

    @functools.partial(
        pl.kernel,
        out_type=[
            jax.ShapeDtypeStruct((qn, kk), jnp.float32),
            jax.ShapeDtypeStruct((qn, kk), jnp.int32),
            jax.ShapeDtypeStruct((qn, kk, d), jnp.float32),
        ],
        mesh=mesh,
        compiler_params=pltpu.CompilerParams(needs_layout_passes=False),
        scratch_types=[
            pltpu.VMEM((g,), jnp.float32),
            pltpu.VMEM((rpad,), jnp.float32),
            pltpu.VMEM((spad,), jnp.float32),
            pltpu.VMEM((kk,), jnp.float32),
            pltpu.VMEM((kk, _L), jnp.float32),
            pltpu.VMEM((cchunks,), jnp.float32),
            pltpu.VMEM((16,), jnp.float32),
            pltpu.VMEM((kk,), jnp.float32),
            pltpu.VMEM((kk,), jnp.int32),
            pltpu.VMEM((kk, d), jnp.float32),
            pltpu.VMEM((16,), jnp.float32),
            pltpu.SMEM((kk,), jnp.int32),
            pltpu.SemaphoreType.DMA,
        ],
    )
    def body(mx_hbm, scores_hbm, keys_hbm, shift_hbm, tv_hbm, ti_hbm, nb_hbm,
             mx_v, rowmax_v, sup_v, gsel_v, cand_v, crow_v, csup_v,
             vals_v, idx_v, nbr_v, shift_v, gs_s, sem):
        wid = lax.axis_index("s") * 2 + lax.axis_index("c")
        pltpu.sync_copy(mx_hbm.at[wid], mx_v)
        pltpu.sync_copy(shift_hbm, shift_v)
        shift = _i32(jnp.max(shift_v[...]))
        iota = _iota16()
        fiota = lax.convert_element_type(iota, jnp.float32)

        for vb in range(rpad // 16):
            rowmax_v[pl.ds(vb * 16, 16)] = jnp.full((16,), _NEG, jnp.float32)

        def rm_body(vb, _):
            vec = jnp.full((16,), _NEG, jnp.float32)
            for l in range(16):
                m = jnp.max(mx_v[pl.ds(vb * 256 + l * 16, 16)])
                vec = jnp.where(iota == l, m, vec)
            rowmax_v[pl.ds(vb * 16, 16)] = vec
            return 0

        lax.fori_loop(0, rows // 16, rm_body, 0)
        if rows % 16:
            vb0 = rows // 16
            vec = jnp.full((16,), _NEG, jnp.float32)
            for l in range(rows % 16):
                m = jnp.max(mx_v[pl.ds(vb0 * 256 + l * 16, 16)])
                vec = jnp.where(iota == l, m, vec)
            rowmax_v[pl.ds(vb0 * 16, 16)] = vec

        for vb in range(spad // 16):
            sup_v[pl.ds(vb * 16, 16)] = jnp.full((16,), _NEG, jnp.float32)

        def sup_body(vb, _):
            _vset(sup_v, vb, jnp.max(rowmax_v[pl.ds(vb * 16, 16)]))
            return 0

        lax.fori_loop(0, nsup, sup_body, 0)

        def sel_body(j, _):
            s0 = sup_v[pl.ds(0, 16)]
            s1 = sup_v[pl.ds(16, 16)]
            m0 = jnp.max(jnp.maximum(s0, s1))
            vb = _i32(jnp.minimum(
                jnp.min(jnp.where(s0 == m0, fiota, _BIG)),
                jnp.min(jnp.where(s1 == m0, fiota + 16.0, _BIG))))
            rv = rowmax_v[pl.ds(vb * 16, 16)]
            r = vb * 16 + _i32(jnp.min(jnp.where(rv == m0, fiota, _BIG)))
            w = mx_v[pl.ds(r * 16, 16)]
            lane = _i32(jnp.min(jnp.where(w == m0, fiota, _BIG)))
            gid = r * 16 + lane
            w2 = jnp.where(iota == lane, _NEG, w)
            mx_v[pl.ds(r * 16, 16)] = w2
            rv2 = jnp.where(iota == r % 16, jnp.max(w2), rv)
            rowmax_v[pl.ds(vb * 16, 16)] = rv2
            _vset(sup_v, vb, jnp.max(rv2))
            _vset(gsel_v, j, lax.convert_element_type(gid, jnp.float32))
            return 0

        lax.fori_loop(0, kk, sel_body, 0)

        def sort_body(a, _):
            g0 = gsel_v[pl.ds(0, 16)]
            g1 = gsel_v[pl.ds(16, 16)]
            m = jnp.minimum(jnp.min(g0), jnp.min(g1))
            p = _i32(jnp.minimum(
                jnp.min(jnp.where(g0 == m, fiota, _BIG)),
                jnp.min(jnp.where(g1 == m, fiota + 16.0, _BIG))))
            mi = _i32(m)
            _vset(gsel_v, p, _BIG)
            gs_s[a] = mi
            return 0

        lax.fori_loop(0, kk, sort_body, 0)

        for c in range(0, kk, 8):
            cps = [pltpu.async_copy(
                       scores_hbm.at[wid, pl.ds(gs_s[a] * _L, _L)],
                       cand_v.at[a], sem)
                   for a in range(c, min(c + 8, kk))]
            for cp in cps:
                cp.wait()

        def cr_body(vb, _):
            vec = jnp.full((16,), _NEG, jnp.float32)
            for l in range(16):
                m = jnp.max(cand_v[vb * 2 + (l // 8), pl.ds((l % 8) * 16, 16)])
                vec = jnp.where(iota == l, m, vec)
            crow_v[pl.ds(vb * 16, 16)] = vec
            _vset(csup_v, vb, jnp.max(vec))
            return 0

        lax.fori_loop(0, csup_n, cr_body, 0)

        def fin_body(j, _):
            c0 = csup_v[...]
            m0 = jnp.max(c0)
            vb = _i32(jnp.min(jnp.where(c0 == m0, fiota, _BIG)))
            rv = crow_v[pl.ds(vb * 16, 16)]
            t = vb * 16 + _i32(jnp.min(jnp.where(rv == m0, fiota, _BIG)))
            a = t // 8
            sub = (t % 8) * 16
            w = cand_v[a, pl.ds(sub, 16)]
            lane = _i32(jnp.min(jnp.where(w == m0, fiota, _BIG)))
            p = t * 16 + lane
            glob = gs_s[p // _L] * _L + p % _L + shift
            w2 = jnp.where(iota == lane, _NEG, w)
            cand_v[a, pl.ds(sub, 16)] = w2
            rv2 = jnp.where(iota == t % 16, jnp.max(w2), rv)
            crow_v[pl.ds(vb * 16, 16)] = rv2
            _vset(csup_v, vb, jnp.max(rv2))
            _vset(vals_v, j, m0)
            _vset(idx_v, j, glob)
            return 0

        lax.fori_loop(0, kk, fin_body, 0)

        pltpu.async_copy(keys_hbm.at[idx_v], nbr_v, sem).wait()
        pltpu.sync_copy(vals_v, tv_hbm.at[wid])
        pltpu.sync_copy(idx_v, ti_hbm.at[wid])
        pltpu.sync_copy(nbr_v, nb_hbm.at[wid])

    return body(mx, scores, keys, shift)


def kernel(queries, keys, k):
    q = queries.shape[0]
    kk = q
    scores, mx = _phase1(queries, keys)
    shift = jnp.full((16,), k - kk, jnp.float32)
    top_vals, top_idx, neighbours = _sc_phase2(mx, scores, keys, shift, kk)
    return top_vals, top_idx, neighbours


# speedup vs baseline: 10.7732x; 1.0291x over previous
import functools

import jax
import jax.numpy as jnp
from jax import lax
from jax.experimental import pallas as pl
from jax.experimental.pallas import tpu as pltpu
from jax.experimental.pallas import tpu_sc as plsc

_B = 32768
_L = 128
_BIG = float(1 << 30)
_NEG = float("-inf")


def _i32(x):
    return lax.convert_element_type(x, jnp.int32)


def _p1_body(nkeys, q_ref, k_ref, s_ref, m_ref):
    i = pl.program_id(0)
    s = lax.dot_general(q_ref[...], k_ref[...], (((1,), (1,)), ((), ())),
                        preferred_element_type=jnp.float32)
    col = i * _B + lax.broadcasted_iota(jnp.int32, s.shape, 1)
    s = jnp.where(col < nkeys, s, _NEG)
    s_ref[...] = s
    m_ref[...] = jnp.max(s.reshape(s.shape[0], _B // _L, _L), axis=2)


def _phase1(queries, keys):
    q, d = queries.shape
    n = keys.shape[0]
    nb = pl.cdiv(n, _B)
    npad = nb * _B
    return pl.pallas_call(
        functools.partial(_p1_body, n),
        grid=(nb,),
        in_specs=[
            pl.BlockSpec((q, d), lambda i: (0, 0)),
            pl.BlockSpec((_B, d), lambda i: (i, 0)),
        ],
        out_specs=[
            pl.BlockSpec((q, _B), lambda i: (0, i)),
            pl.BlockSpec((q, _B // _L), lambda i: (0, i)),
        ],
        out_shape=[
            jax.ShapeDtypeStruct((q, npad), jnp.float32),
            jax.ShapeDtypeStruct((q, npad // _L), jnp.float32),
        ],
    )(queries, keys)


def _iota16():
    return lax.broadcasted_iota(jnp.int32, (16,), 0)


def _vset(ref, pos, val):
    base = (pos // 16) * 16
    v = ref[pl.ds(base, 16)]
    ref[pl.ds(base, 16)] = jnp.where(_iota16() == pos % 16, val, v)


def _sc_phase2(mx, scores, keys, shift, kk):
    qn, g = mx.shape
    nkr = keys.shape[0]
    d = keys.shape[1]
    rows = g // 16
    rpad = ((rows + 15) // 16) * 16
    nsup = rpad // 16
    spad = ((nsup + 15) // 16) * 16
    ncand = kk * _L
    cchunks = ncand // 16
    csup_n = cchunks // 16
    mesh = plsc.VectorSubcoreMesh(core_axis_name="c", subcore_axis_name="s")

    @functools.partial(
        pl.kernel,
        out_type=[
            jax.ShapeDtypeStruct((qn, kk), jnp.float32),
            jax.ShapeDtypeStruct((qn, kk), jnp.int32),
            jax.ShapeDtypeStruct((qn, kk, d), jnp.float32),
        ],
        mesh=mesh,
        compiler_params=pltpu.CompilerParams(needs_layout_passes=False),
        scratch_types=[
            pltpu.VMEM((g,), jnp.float32),
            pltpu.VMEM((rpad,), jnp.float32),
            pltpu.VMEM((spad,), jnp.float32),
            pltpu.VMEM((kk,), jnp.float32),
            pltpu.VMEM((kk, _L), jnp.float32),
            pltpu.VMEM((cchunks,), jnp.float32),
            pltpu.VMEM((16,), jnp.float32),
            pltpu.VMEM((kk,), jnp.float32),
            pltpu.VMEM((kk,), jnp.int32),
            pltpu.VMEM((kk, d), jnp.float32),
            pltpu.VMEM((16,), jnp.float32),
            pltpu.SMEM((kk,), jnp.int32),
            pltpu.SemaphoreType.DMA,
        ],
    )
    def body(mx_hbm, scores_hbm, keys_hbm, shift_hbm, tv_hbm, ti_hbm, nb_hbm,
             mx_v, rowmax_v, sup_v, gsel_v, cand_v, crow_v, csup_v,
             vals_v, idx_v, nbr_v, shift_v, gs_s, sem):
        wid = lax.axis_index("s") * 2 + lax.axis_index("c")
        pltpu.sync_copy(mx_hbm.at[wid], mx_v)
        pltpu.sync_copy(shift_hbm, shift_v)
        shift = _i32(jnp.max(shift_v[...]))
        iota = _iota16()
        fiota = lax.convert_element_type(iota, jnp.float32)

        for vb in range(rpad // 16):
            rowmax_v[pl.ds(vb * 16, 16)] = jnp.full((16,), _NEG, jnp.float32)

        def rm_body(vb, _):
            vec = jnp.full((16,), _NEG, jnp.float32)
            for l in range(16):
                m = jnp.max(mx_v[pl.ds(vb * 256 + l * 16, 16)])
                vec = jnp.where(iota == l, m, vec)
            rowmax_v[pl.ds(vb * 16, 16)] = vec
            return 0

        lax.fori_loop(0, rows // 16, rm_body, 0)
        if rows % 16:
            vb0 = rows // 16
            vec = jnp.full((16,), _NEG, jnp.float32)
            for l in range(rows % 16):
                m = jnp.max(mx_v[pl.ds(vb0 * 256 + l * 16, 16)])
                vec = jnp.where(iota == l, m, vec)
            rowmax_v[pl.ds(vb0 * 16, 16)] = vec

        for vb in range(spad // 16):
            sup_v[pl.ds(vb * 16, 16)] = jnp.full((16,), _NEG, jnp.float32)

        def sup_body(vb, _):
            _vset(sup_v, vb, jnp.max(rowmax_v[pl.ds(vb * 16, 16)]))
            return 0

        lax.fori_loop(0, nsup, sup_body, 0)

        def sel_body(j, _):
            s0 = sup_v[pl.ds(0, 16)]
            s1 = sup_v[pl.ds(16, 16)]
            m0 = jnp.max(jnp.maximum(s0, s1))
            vb = _i32(jnp.minimum(
                jnp.min(jnp.where(s0 == m0, fiota, _BIG)),
                jnp.min(jnp.where(s1 == m0, fiota + 16.0, _BIG))))
            rv = rowmax_v[pl.ds(vb * 16, 16)]
            r = vb * 16 + _i32(jnp.min(jnp.where(rv == m0, fiota, _BIG)))
            w = mx_v[pl.ds(r * 16, 16)]
            lane = _i32(jnp.min(jnp.where(w == m0, fiota, _BIG)))
            gid = r * 16 + lane
            w2 = jnp.where(iota == lane, _NEG, w)
            mx_v[pl.ds(r * 16, 16)] = w2
            rv2 = jnp.where(iota == r % 16, jnp.max(w2), rv)
            rowmax_v[pl.ds(vb * 16, 16)] = rv2
            _vset(sup_v, vb, jnp.max(rv2))
            _vset(gsel_v, j, lax.convert_element_type(gid, jnp.float32))
            return 0

        lax.fori_loop(0, kk, sel_body, 0)

        def sort_body(a, _):
            g0 = gsel_v[pl.ds(0, 16)]
            g1 = gsel_v[pl.ds(16, 16)]
            m = jnp.minimum(jnp.min(g0), jnp.min(g1))
            p = _i32(jnp.minimum(
                jnp.min(jnp.where(g0 == m, fiota, _BIG)),
                jnp.min(jnp.where(g1 == m, fiota + 16.0, _BIG))))
            mi = _i32(m)
            _vset(gsel_v, p, _BIG)
            gs_s[a] = mi
            return 0

        lax.fori_loop(0, kk, sort_body, 0)

        for c in range(0, kk, 8):
            cps = [pltpu.async_copy(
                       scores_hbm.at[wid, pl.ds(gs_s[a] * _L, _L)],
                       cand_v.at[a], sem)
                   for a in range(c, min(c + 8, kk))]
            for cp in cps:
                cp.wait()

        def cr_body(vb, _):
            vec = jnp.full((16,), _NEG, jnp.float32)
            for l in range(16):
                m = jnp.max(cand_v[vb * 2 + (l // 8), pl.ds((l % 8) * 16, 16)])
                vec = jnp.where(iota == l, m, vec)
            crow_v[pl.ds(vb * 16, 16)] = vec
            _vset(csup_v, vb, jnp.max(vec))
            return 0

        lax.fori_loop(0, csup_n, cr_body, 0)

        def fin_body(j, _):
            c0 = csup_v[...]
            m0 = jnp.max(c0)
            vb = _i32(jnp.min(jnp.where(c0 == m0, fiota, _BIG)))
            rv = crow_v[pl.ds(vb * 16, 16)]
            t = vb * 16 + _i32(jnp.min(jnp.where(rv == m0, fiota, _BIG)))
            a = t // 8
            sub = (t % 8) * 16
            w = cand_v[a, pl.ds(sub, 16)]
            lane = _i32(jnp.min(jnp.where(w == m0, fiota, _BIG)))
            p = t * 16 + lane
            glob = gs_s[p // _L] * _L + p % _L + shift
            w2 = jnp.where(iota == lane, _NEG, w)
            cand_v[a, pl.ds(sub, 16)] = w2
            rv2 = jnp.where(iota == t % 16, jnp.max(w2), rv)
            crow_v[pl.ds(vb * 16, 16)] = rv2
            _vset(csup_v, vb, jnp.max(rv2))
            _vset(vals_v, j, m0)
            _vset(idx_v, j, glob)
            return 0

        lax.fori_loop(0, kk, fin_body, 0)

        pltpu.async_copy(keys_hbm.at[idx_v], nbr_v, sem).wait()
        pltpu.sync_copy(vals_v, tv_hbm.at[wid])
        pltpu.sync_copy(idx_v, ti_hbm.at[wid])
        pltpu.sync_copy(nbr_v, nb_hbm.at[wid])

    return body(mx, scores, keys, shift)


def kernel(queries, keys, k):
    q = queries.shape[0]
    kk = q
    scores, mx = _phase1(queries, keys)
    shift = jnp.full((16,), k - kk, jnp.float32)
    top_vals, top_idx, neighbours = _sc_phase2(mx, scores, keys, shift, kk)
    return top_vals, top_idx, neighbours


# speedup vs baseline: 10.8476x; 1.0069x over previous
import functools

import jax
import jax.numpy as jnp
from jax import lax
from jax.experimental import pallas as pl
from jax.experimental.pallas import tpu as pltpu
from jax.experimental.pallas import tpu_sc as plsc

_B = 32768
_L = 128
_BIG = float(1 << 30)
_NEG = float("-inf")


def _i32(x):
    return lax.convert_element_type(x, jnp.int32)


def _p1_body(nkeys, q_ref, k_ref, s_ref, m_ref):
    i = pl.program_id(0)
    s = lax.dot_general(q_ref[...], k_ref[...], (((1,), (1,)), ((), ())),
                        preferred_element_type=jnp.float32)
    col = i * _B + lax.broadcasted_iota(jnp.int32, s.shape, 1)
    s = jnp.where(col < nkeys, s, _NEG)
    s_ref[...] = s
    m_ref[...] = jnp.max(s.reshape(s.shape[0], _B // _L, _L), axis=2)


def _phase1(queries, keys):
    q, d = queries.shape
    n = keys.shape[0]
    nb = pl.cdiv(n, _B)
    npad = nb * _B
    return pl.pallas_call(
        functools.partial(_p1_body, n),
        grid=(nb,),
        in_specs=[
            pl.BlockSpec((q, d), lambda i: (0, 0)),
            pl.BlockSpec((_B, d), lambda i: (i, 0)),
        ],
        out_specs=[
            pl.BlockSpec((q, _B), lambda i: (0, i)),
            pl.BlockSpec((q, _B // _L), lambda i: (0, i)),
        ],
        out_shape=[
            jax.ShapeDtypeStruct((q, npad), jnp.float32),
            jax.ShapeDtypeStruct((q, npad // _L), jnp.float32),
        ],
    )(queries, keys)


def _iota16():
    return lax.broadcasted_iota(jnp.int32, (16,), 0)


def _vset(ref, pos, val):
    base = (pos // 16) * 16
    v = ref[pl.ds(base, 16)]
    ref[pl.ds(base, 16)] = jnp.where(_iota16() == pos % 16, val, v)


def _sc_phase2(mx, scores, keys, shift, kk):
    qn, g = mx.shape
    nkr = keys.shape[0]
    d = keys.shape[1]
    rows = g // 16
    rpad = ((rows + 15) // 16) * 16
    nsup = rpad // 16
    spad = ((nsup + 15) // 16) * 16
    ncand = kk * _L
    cchunks = ncand // 16
    csup_n = cchunks // 16
    mesh = plsc.VectorSubcoreMesh(core_axis_name="c", subcore_axis_name="s")

    @functools.partial(
        pl.kernel,
        out_type=[
            jax.ShapeDtypeStruct((qn, kk), jnp.float32),
            jax.ShapeDtypeStruct((qn, kk), jnp.int32),
            jax.ShapeDtypeStruct((qn, kk, d), jnp.float32),
        ],
        mesh=mesh,
        compiler_params=pltpu.CompilerParams(needs_layout_passes=False),
        scratch_types=[
            pltpu.VMEM((g,), jnp.float32),
            pltpu.VMEM((rpad,), jnp.float32),
            pltpu.VMEM((spad,), jnp.float32),
            pltpu.VMEM((kk,), jnp.float32),
            pltpu.VMEM((kk, _L), jnp.float32),
            pltpu.VMEM((cchunks,), jnp.float32),
            pltpu.VMEM((16,), jnp.float32),
            pltpu.VMEM((kk,), jnp.float32),
            pltpu.VMEM((kk,), jnp.int32),
            pltpu.VMEM((kk, d), jnp.float32),
            pltpu.VMEM((16,), jnp.float32),
            pltpu.SMEM((kk,), jnp.int32),
            pltpu.SemaphoreType.DMA,
        ],
    )
    def body(mx_hbm, scores_hbm, keys_hbm, shift_hbm,
             tv_hbm, ti_hbm, nb_hbm,
             mx_v, rowmax_v, sup_v, gsel_v, cand_v, crow_v, csup_v,
             vals_v, idx_v, nbr_v, shift_v, gs_s, sem):
        wid = lax.axis_index("s") * 2 + lax.axis_index("c")
        pltpu.sync_copy(mx_hbm.at[wid], mx_v)
        pltpu.sync_copy(shift_hbm, shift_v)
        shift = _i32(jnp.max(shift_v[...]))
        iota = _iota16()
        fiota = lax.convert_element_type(iota, jnp.float32)

        def rm_body(vb, _):
            vec = jnp.full((16,), _NEG, jnp.float32)
            for l in range(16):
                m = jnp.max(mx_v[pl.ds(vb * 256 + l * 16, 16)])
                vec = jnp.where(iota == l, m, vec)
            rowmax_v[pl.ds(vb * 16, 16)] = vec
            return 0

        lax.fori_loop(0, rows // 16, rm_body, 0)

        for vb in range(spad // 16):
            sup_v[pl.ds(vb * 16, 16)] = jnp.full((16,), _NEG, jnp.float32)

        def sup_body(vb, _):
            _vset(sup_v, vb, jnp.max(rowmax_v[pl.ds(vb * 16, 16)]))
            return 0

        lax.fori_loop(0, nsup, sup_body, 0)

        def sel_body(j, _):
            s0 = sup_v[pl.ds(0, 16)]
            s1 = sup_v[pl.ds(16, 16)]
            m0 = jnp.max(jnp.maximum(s0, s1))
            vb = _i32(jnp.minimum(
                jnp.min(jnp.where(s0 == m0, fiota, _BIG)),
                jnp.min(jnp.where(s1 == m0, fiota + 16.0, _BIG))))
            rv = rowmax_v[pl.ds(vb * 16, 16)]
            r = vb * 16 + _i32(jnp.min(jnp.where(rv == m0, fiota, _BIG)))
            w = mx_v[pl.ds(r * 16, 16)]
            lane = _i32(jnp.min(jnp.where(w == m0, fiota, _BIG)))
            gid = r * 16 + lane
            w2 = jnp.where(iota == lane, _NEG, w)
            mx_v[pl.ds(r * 16, 16)] = w2
            rv2 = jnp.where(iota == r % 16, jnp.max(w2), rv)
            rowmax_v[pl.ds(vb * 16, 16)] = rv2
            _vset(sup_v, vb, jnp.max(rv2))
            _vset(gsel_v, j, lax.convert_element_type(gid, jnp.float32))
            return 0

        lax.fori_loop(0, kk, sel_body, 0)

        def sort_body(a, _):
            g0 = gsel_v[pl.ds(0, 16)]
            g1 = gsel_v[pl.ds(16, 16)]
            m = jnp.minimum(jnp.min(g0), jnp.min(g1))
            p = _i32(jnp.minimum(
                jnp.min(jnp.where(g0 == m, fiota, _BIG)),
                jnp.min(jnp.where(g1 == m, fiota + 16.0, _BIG))))
            mi = _i32(m)
            _vset(gsel_v, p, _BIG)
            gs_s[a] = mi
            return 0

        lax.fori_loop(0, kk, sort_body, 0)

        cps = [pltpu.async_copy(
                   scores_hbm.at[wid, pl.ds(gs_s[a] * _L, _L)],
                   cand_v.at[a], sem)
               for a in range(kk)]
        for cp in cps:
            cp.wait()

        def cr_body(vb, _):
            vec = jnp.full((16,), _NEG, jnp.float32)
            for l in range(16):
                m = jnp.max(cand_v[vb * 2 + (l // 8), pl.ds((l % 8) * 16, 16)])
                vec = jnp.where(iota == l, m, vec)
            crow_v[pl.ds(vb * 16, 16)] = vec
            _vset(csup_v, vb, jnp.max(vec))
            return 0

        lax.fori_loop(0, csup_n, cr_body, 0)

        def fin_body(j, _):
            c0 = csup_v[...]
            m0 = jnp.max(c0)
            vb = _i32(jnp.min(jnp.where(c0 == m0, fiota, _BIG)))
            rv = crow_v[pl.ds(vb * 16, 16)]
            t = vb * 16 + _i32(jnp.min(jnp.where(rv == m0, fiota, _BIG)))
            a = t // 8
            sub = (t % 8) * 16
            w = cand_v[a, pl.ds(sub, 16)]
            lane = _i32(jnp.min(jnp.where(w == m0, fiota, _BIG)))
            p = t * 16 + lane
            glob = gs_s[p // _L] * _L + p % _L + shift
            w2 = jnp.where(iota == lane, _NEG, w)
            cand_v[a, pl.ds(sub, 16)] = w2
            rv2 = jnp.where(iota == t % 16, jnp.max(w2), rv)
            crow_v[pl.ds(vb * 16, 16)] = rv2
            _vset(csup_v, vb, jnp.max(rv2))
            _vset(vals_v, j, m0)
            _vset(idx_v, j, glob)
            return 0

        lax.fori_loop(0, kk, fin_body, 0)

        pltpu.async_copy(keys_hbm.at[idx_v], nbr_v, sem).wait()
        pltpu.sync_copy(vals_v, tv_hbm.at[wid])
        pltpu.sync_copy(idx_v, ti_hbm.at[wid])
        pltpu.sync_copy(nbr_v, nb_hbm.at[wid])

    return body(mx, scores, keys, shift)


def kernel(queries, keys, k):
    q = queries.shape[0]
    kk = q
    scores, mx = _phase1(queries, keys)
    shift = jnp.full((16,), k - kk, jnp.float32)
    top_vals, top_idx, neighbours = _sc_phase2(mx, scores, keys, shift, kk)
    return top_vals, top_idx, neighbours


# speedup vs baseline: 10.8689x; 1.0020x over previous
import functools

import jax
import jax.numpy as jnp
from jax import lax
from jax.experimental import pallas as pl
from jax.experimental.pallas import tpu as pltpu
from jax.experimental.pallas import tpu_sc as plsc

_B = 32768
_L = 128
_BIG = float(1 << 30)
_NEG = float("-inf")


def _i32(x):
    return lax.convert_element_type(x, jnp.int32)


def _p1_body(nkeys, q_ref, k_ref, kval_ref, s_ref, m_ref, sh_ref):
    i = pl.program_id(0)
    s = lax.dot_general(q_ref[...], k_ref[...], (((1,), (1,)), ((), ())),
                        preferred_element_type=jnp.float32)
    col = i * _B + lax.broadcasted_iota(jnp.int32, s.shape, 1)
    s = jnp.where(col < nkeys, s, _NEG)
    s_ref[...] = s
    m_ref[...] = jnp.max(s.reshape(s.shape[0], _B // _L, _L), axis=2)
    shv = lax.convert_element_type(kval_ref[0, 0] - s.shape[0], jnp.float32)
    sh_ref[...] = jnp.full((16,), shv, jnp.float32)


def _phase1(queries, keys, kval):
    q, d = queries.shape
    n = keys.shape[0]
    nb = pl.cdiv(n, _B)
    npad = nb * _B
    return pl.pallas_call(
        functools.partial(_p1_body, n),
        grid=(nb,),
        in_specs=[
            pl.BlockSpec((q, d), lambda i: (0, 0)),
            pl.BlockSpec((_B, d), lambda i: (i, 0)),
            pl.BlockSpec(memory_space=pltpu.SMEM),
        ],
        out_specs=[
            pl.BlockSpec((q, _B), lambda i: (0, i)),
            pl.BlockSpec((q, _B // _L), lambda i: (0, i)),
            pl.BlockSpec((16,), lambda i: (0,)),
        ],
        out_shape=[
            jax.ShapeDtypeStruct((q, npad), jnp.float32),
            jax.ShapeDtypeStruct((q, npad // _L), jnp.float32),
            jax.ShapeDtypeStruct((16,), jnp.float32),
        ],
    )(queries, keys, kval)


def _iota16():
    return lax.broadcasted_iota(jnp.int32, (16,), 0)


def _vset(ref, pos, val):
    base = (pos // 16) * 16
    v = ref[pl.ds(base, 16)]
    ref[pl.ds(base, 16)] = jnp.where(_iota16() == pos % 16, val, v)


def _sc_phase2(mx, scores, keys, shift, kk):
    qn, g = mx.shape
    nkr = keys.shape[0]
    d = keys.shape[1]
    rows = g // 16
    rpad = ((rows + 15) // 16) * 16
    nsup = rpad // 16
    spad = ((nsup + 15) // 16) * 16
    ncand = kk * _L
    cchunks = ncand // 16
    csup_n = cchunks // 16
    mesh = plsc.VectorSubcoreMesh(core_axis_name="c", subcore_axis_name="s")

    @functools.partial(
        pl.kernel,
        out_type=[
            jax.ShapeDtypeStruct((qn, kk), jnp.float32),
            jax.ShapeDtypeStruct((qn, kk), jnp.int32),
            jax.ShapeDtypeStruct((qn, kk, d), jnp.float32),
        ],
        mesh=mesh,
        compiler_params=pltpu.CompilerParams(needs_layout_passes=False),
        scratch_types=[
            pltpu.VMEM((g,), jnp.float32),
            pltpu.VMEM((rpad,), jnp.float32),
            pltpu.VMEM((spad,), jnp.float32),
            pltpu.VMEM((kk,), jnp.float32),
            pltpu.VMEM((kk, _L), jnp.float32),
            pltpu.VMEM((cchunks,), jnp.float32),
            pltpu.VMEM((16,), jnp.float32),
            pltpu.VMEM((kk,), jnp.float32),
            pltpu.VMEM((kk,), jnp.int32),
            pltpu.VMEM((kk, d), jnp.float32),
            pltpu.VMEM((16,), jnp.float32),
            pltpu.SMEM((kk,), jnp.int32),
            pltpu.SemaphoreType.DMA,
        ],
    )
    def body(mx_hbm, scores_hbm, keys_hbm, shift_hbm,
             tv_hbm, ti_hbm, nb_hbm,
             mx_v, rowmax_v, sup_v, gsel_v, cand_v, crow_v, csup_v,
             vals_v, idx_v, nbr_v, shift_v, gs_s, sem):
        wid = lax.axis_index("s") * 2 + lax.axis_index("c")
        pltpu.sync_copy(mx_hbm.at[wid], mx_v)
        pltpu.sync_copy(shift_hbm, shift_v)
        shift = _i32(jnp.max(shift_v[...]))
        iota = _iota16()
        fiota = lax.convert_element_type(iota, jnp.float32)

        def rm_body(vb, _):
            vec = jnp.full((16,), _NEG, jnp.float32)
            for l in range(16):
                m = jnp.max(mx_v[pl.ds(vb * 256 + l * 16, 16)])
                vec = jnp.where(iota == l, m, vec)
            rowmax_v[pl.ds(vb * 16, 16)] = vec
            return 0

        lax.fori_loop(0, rows // 16, rm_body, 0)

        for vb in range(spad // 16):
            sup_v[pl.ds(vb * 16, 16)] = jnp.full((16,), _NEG, jnp.float32)

        def sup_body(vb, _):
            _vset(sup_v, vb, jnp.max(rowmax_v[pl.ds(vb * 16, 16)]))
            return 0

        lax.fori_loop(0, nsup, sup_body, 0)

        def sel_body(j, _):
            s0 = sup_v[pl.ds(0, 16)]
            s1 = sup_v[pl.ds(16, 16)]
            m0 = jnp.max(jnp.maximum(s0, s1))
            vb = _i32(jnp.minimum(
                jnp.min(jnp.where(s0 == m0, fiota, _BIG)),
                jnp.min(jnp.where(s1 == m0, fiota + 16.0, _BIG))))
            rv = rowmax_v[pl.ds(vb * 16, 16)]
            r = vb * 16 + _i32(jnp.min(jnp.where(rv == m0, fiota, _BIG)))
            w = mx_v[pl.ds(r * 16, 16)]
            lane = _i32(jnp.min(jnp.where(w == m0, fiota, _BIG)))
            gid = r * 16 + lane
            w2 = jnp.where(iota == lane, _NEG, w)
            mx_v[pl.ds(r * 16, 16)] = w2
            rv2 = jnp.where(iota == r % 16, jnp.max(w2), rv)
            rowmax_v[pl.ds(vb * 16, 16)] = rv2
            _vset(sup_v, vb, jnp.max(rv2))
            _vset(gsel_v, j, lax.convert_element_type(gid, jnp.float32))
            return 0

        lax.fori_loop(0, kk, sel_body, 0)

        def sort_body(a, _):
            g0 = gsel_v[pl.ds(0, 16)]
            g1 = gsel_v[pl.ds(16, 16)]
            m = jnp.minimum(jnp.min(g0), jnp.min(g1))
            p = _i32(jnp.minimum(
                jnp.min(jnp.where(g0 == m, fiota, _BIG)),
                jnp.min(jnp.where(g1 == m, fiota + 16.0, _BIG))))
            mi = _i32(m)
            _vset(gsel_v, p, _BIG)
            gs_s[a] = mi
            return 0

        lax.fori_loop(0, kk, sort_body, 0)

        cps = [pltpu.async_copy(
                   scores_hbm.at[wid, pl.ds(gs_s[a] * _L, _L)],
                   cand_v.at[a], sem)
               for a in range(kk)]
        for cp in cps:
            cp.wait()

        def cr_body(vb, _):
            vec = jnp.full((16,), _NEG, jnp.float32)
            for l in range(16):
                m = jnp.max(cand_v[vb * 2 + (l // 8), pl.ds((l % 8) * 16, 16)])
                vec = jnp.where(iota == l, m, vec)
            crow_v[pl.ds(vb * 16, 16)] = vec
            _vset(csup_v, vb, jnp.max(vec))
            return 0

        lax.fori_loop(0, csup_n, cr_body, 0)

        def fin_body(j, _):
            c0 = csup_v[...]
            m0 = jnp.max(c0)
            vb = _i32(jnp.min(jnp.where(c0 == m0, fiota, _BIG)))
            rv = crow_v[pl.ds(vb * 16, 16)]
            t = vb * 16 + _i32(jnp.min(jnp.where(rv == m0, fiota, _BIG)))
            a = t // 8
            sub = (t % 8) * 16
            w = cand_v[a, pl.ds(sub, 16)]
            lane = _i32(jnp.min(jnp.where(w == m0, fiota, _BIG)))
            p = t * 16 + lane
            glob = gs_s[p // _L] * _L + p % _L + shift
            w2 = jnp.where(iota == lane, _NEG, w)
            cand_v[a, pl.ds(sub, 16)] = w2
            rv2 = jnp.where(iota == t % 16, jnp.max(w2), rv)
            crow_v[pl.ds(vb * 16, 16)] = rv2
            _vset(csup_v, vb, jnp.max(rv2))
            _vset(vals_v, j, m0)
            _vset(idx_v, j, glob)
            return 0

        lax.fori_loop(0, kk, fin_body, 0)

        pltpu.async_copy(keys_hbm.at[idx_v], nbr_v, sem).wait()
        pltpu.sync_copy(vals_v, tv_hbm.at[wid])
        pltpu.sync_copy(idx_v, ti_hbm.at[wid])
        pltpu.sync_copy(nbr_v, nb_hbm.at[wid])

    return body(mx, scores, keys, shift)


def kernel(queries, keys, k):
    q = queries.shape[0]
    kk = q
    kval = jnp.asarray(k, jnp.int32).reshape(1, 1)
    scores, mx, shift = _phase1(queries, keys, kval)
    top_vals, top_idx, neighbours = _sc_phase2(mx, scores, keys, shift, kk)
    return top_vals, top_idx, neighbours


# speedup vs baseline: 10.8809x; 1.0011x over previous
import functools

import jax
import jax.numpy as jnp
from jax import lax
from jax.experimental import pallas as pl
from jax.experimental.pallas import tpu as pltpu
from jax.experimental.pallas import tpu_sc as plsc

_B = 32768
_L = 128
_BIG = float(1 << 30)
_NEG = float("-inf")


def _i32(x):
    return lax.convert_element_type(x, jnp.int32)


def _p1_body(nkeys, q_ref, k_ref, kval_ref, s_ref, m_ref, sh_ref):
    i = pl.program_id(0)
    s = lax.dot_general(q_ref[...], k_ref[...], (((1,), (1,)), ((), ())),
                        preferred_element_type=jnp.float32)
    col = i * _B + lax.broadcasted_iota(jnp.int32, s.shape, 1)
    s = jnp.where(col < nkeys, s, _NEG)
    s_ref[...] = s
    m_ref[...] = jnp.max(s.reshape(s.shape[0], _B // _L, _L), axis=2)
    shv = lax.convert_element_type(kval_ref[0, 0] - s.shape[0], jnp.float32)
    sh_ref[...] = jnp.full((16,), shv, jnp.float32)


def _phase1(queries, keys, kval):
    q, d = queries.shape
    n = keys.shape[0]
    nb = pl.cdiv(n, _B)
    npad = nb * _B
    return pl.pallas_call(
        functools.partial(_p1_body, n),
        grid=(nb,),
        in_specs=[
            pl.BlockSpec((q, d), lambda i: (0, 0)),
            pl.BlockSpec((_B, d), lambda i: (i, 0)),
            pl.BlockSpec(memory_space=pltpu.SMEM),
        ],
        out_specs=[
            pl.BlockSpec((q, _B), lambda i: (0, i)),
            pl.BlockSpec((q, _B // _L), lambda i: (0, i)),
            pl.BlockSpec((16,), lambda i: (0,)),
        ],
        out_shape=[
            jax.ShapeDtypeStruct((q, npad), jnp.float32),
            jax.ShapeDtypeStruct((q, npad // _L), jnp.float32),
            jax.ShapeDtypeStruct((16,), jnp.float32),
        ],
    )(queries, keys, kval)


def _iota16():
    return lax.broadcasted_iota(jnp.int32, (16,), 0)


def _vset(ref, pos, val):
    base = (pos // 16) * 16
    v = ref[pl.ds(base, 16)]
    ref[pl.ds(base, 16)] = jnp.where(_iota16() == pos % 16, val, v)


def _sc_phase2(mx, scores, keys, shift, kk):
    qn, g = mx.shape
    nkr = keys.shape[0]
    d = keys.shape[1]
    rows = g // 16
    rpad = ((rows + 15) // 16) * 16
    nsup = rpad // 16
    spad = ((nsup + 15) // 16) * 16
    ncand = kk * _L
    cchunks = ncand // 16
    csup_n = cchunks // 16
    mesh = plsc.VectorSubcoreMesh(core_axis_name="c", subcore_axis_name="s")

    @functools.partial(
        pl.kernel,
        out_type=[
            jax.ShapeDtypeStruct((qn, kk), jnp.float32),
            jax.ShapeDtypeStruct((qn, kk), jnp.int32),
            jax.ShapeDtypeStruct((qn, kk, d), jnp.float32),
        ],
        mesh=mesh,
        compiler_params=pltpu.CompilerParams(needs_layout_passes=False),
        scratch_types=[
            pltpu.VMEM((g,), jnp.float32),
            pltpu.VMEM((rpad,), jnp.float32),
            pltpu.VMEM((spad,), jnp.float32),
            pltpu.VMEM((kk,), jnp.float32),
            pltpu.VMEM((kk, _L), jnp.float32),
            pltpu.VMEM((cchunks,), jnp.float32),
            pltpu.VMEM((16,), jnp.float32),
            pltpu.VMEM((kk,), jnp.float32),
            pltpu.VMEM((kk,), jnp.int32),
            pltpu.VMEM((kk, d), jnp.float32),
            pltpu.VMEM((16,), jnp.float32),
            pltpu.SMEM((kk,), jnp.int32),
            pltpu.SemaphoreType.DMA,
        ],
    )
    def body(mx_hbm, scores_hbm, keys_hbm, shift_hbm,
             tv_hbm, ti_hbm, nb_hbm,
             mx_v, rowmax_v, sup_v, gsel_v, cand_v, crow_v, csup_v,
             vals_v, idx_v, nbr_v, shift_v, gs_s, sem):
        wid = lax.axis_index("s") * 2 + lax.axis_index("c")
        pltpu.sync_copy(mx_hbm.at[wid], mx_v)
        pltpu.sync_copy(shift_hbm, shift_v)
        shift = _i32(jnp.max(shift_v[...]))
        iota = _iota16()
        fiota = lax.convert_element_type(iota, jnp.float32)

        def vm_build(vb, _):
            acc = jnp.full((16,), _NEG, jnp.float32)
            for j in range(16):
                acc = jnp.maximum(acc, mx_v[pl.ds(vb * 256 + j * 16, 16)])
            rowmax_v[pl.ds(vb * 16, 16)] = acc
            _vset(sup_v, vb, jnp.max(acc))
            return 0

        for vb in range(spad // 16):
            sup_v[pl.ds(vb * 16, 16)] = jnp.full((16,), _NEG, jnp.float32)
        lax.fori_loop(0, nsup, vm_build, 0)

        def sel_body(j, _):
            s0 = sup_v[pl.ds(0, 16)]
            s1 = sup_v[pl.ds(16, 16)]
            m0 = jnp.max(jnp.maximum(s0, s1))
            vb = _i32(jnp.min(jnp.minimum(
                jnp.where(s0 == m0, fiota, _BIG),
                jnp.where(s1 == m0, fiota + 16.0, _BIG))))
            off_acc = jnp.full((16,), _BIG, jnp.float32)
            for jj in range(16):
                w = mx_v[pl.ds(vb * 256 + jj * 16, 16)]
                off_acc = jnp.minimum(
                    off_acc, jnp.where(w == m0, jj * 16.0 + fiota, _BIG))
            off = _i32(jnp.min(off_acc))
            gid = vb * 256 + off
            wv = mx_v[pl.ds(vb * 256 + (off // 16) * 16, 16)]
            mx_v[pl.ds(vb * 256 + (off // 16) * 16, 16)] = (
                jnp.where(iota == off % 16, _NEG, wv))
            acc = jnp.full((16,), _NEG, jnp.float32)
            for jj in range(16):
                acc = jnp.maximum(acc, mx_v[pl.ds(vb * 256 + jj * 16, 16)])
            rowmax_v[pl.ds(vb * 16, 16)] = acc
            _vset(sup_v, vb, jnp.max(acc))
            _vset(gsel_v, j, lax.convert_element_type(gid, jnp.float32))
            return 0

        lax.fori_loop(0, kk, sel_body, 0)

        def sort_body(a, _):
            g0 = gsel_v[pl.ds(0, 16)]
            g1 = gsel_v[pl.ds(16, 16)]
            m = jnp.minimum(jnp.min(g0), jnp.min(g1))
            p = _i32(jnp.minimum(
                jnp.min(jnp.where(g0 == m, fiota, _BIG)),
                jnp.min(jnp.where(g1 == m, fiota + 16.0, _BIG))))
            mi = _i32(m)
            _vset(gsel_v, p, _BIG)
            gs_s[a] = mi
            return 0

        lax.fori_loop(0, kk, sort_body, 0)

        cps = [pltpu.async_copy(
                   scores_hbm.at[wid, pl.ds(gs_s[a] * _L, _L)],
                   cand_v.at[a], sem)
               for a in range(kk)]
        for cp in cps:
            cp.wait()

        def cr_body(sb, _):
            acc = jnp.full((16,), _NEG, jnp.float32)
            for j in range(16):
                acc = jnp.maximum(
                    acc, cand_v[sb * 2 + (j // 8), pl.ds((j % 8) * 16, 16)])
            crow_v[pl.ds(sb * 16, 16)] = acc
            _vset(csup_v, sb, jnp.max(acc))
            return 0

        lax.fori_loop(0, csup_n, cr_body, 0)

        def fin_body(j, _):
            c0 = csup_v[...]
            m0 = jnp.max(c0)
            sb = _i32(jnp.min(jnp.where(c0 == m0, fiota, _BIG)))
            off_acc = jnp.full((16,), _BIG, jnp.float32)
            for jj in range(16):
                w = cand_v[sb * 2 + (jj // 8), pl.ds((jj % 8) * 16, 16)]
                off_acc = jnp.minimum(
                    off_acc, jnp.where(w == m0, jj * 16.0 + fiota, _BIG))
            off = _i32(jnp.min(off_acc))
            p = sb * 256 + off
            glob = gs_s[p // _L] * _L + p % _L + shift
            a2 = sb * 2 + off // 128
            sub2 = ((off // 16) % 8) * 16
            wv = cand_v[a2, pl.ds(sub2, 16)]
            cand_v[a2, pl.ds(sub2, 16)] = jnp.where(iota == off % 16, _NEG, wv)
            acc = jnp.full((16,), _NEG, jnp.float32)
            for jj in range(16):
                acc = jnp.maximum(
                    acc, cand_v[sb * 2 + (jj // 8), pl.ds((jj % 8) * 16, 16)])
            crow_v[pl.ds(sb * 16, 16)] = acc
            _vset(csup_v, sb, jnp.max(acc))
            _vset(vals_v, j, m0)
            _vset(idx_v, j, glob)
            return 0

        lax.fori_loop(0, kk, fin_body, 0)

        pltpu.async_copy(keys_hbm.at[idx_v], nbr_v, sem).wait()
        pltpu.sync_copy(vals_v, tv_hbm.at[wid])
        pltpu.sync_copy(idx_v, ti_hbm.at[wid])
        pltpu.sync_copy(nbr_v, nb_hbm.at[wid])

    return body(mx, scores, keys, shift)


def kernel(queries, keys, k):
    q = queries.shape[0]
    kk = q
    kval = jnp.asarray(k, jnp.int32).reshape(1, 1)
    scores, mx, shift = _phase1(queries, keys, kval)
    top_vals, top_idx, neighbours = _sc_phase2(mx, scores, keys, shift, kk)
    return top_vals, top_idx, neighbours
